# baseline (speedup 1.0000x reference)
import functools

import jax
import jax.numpy as jnp
from jax.experimental import pallas as pl
from jax.experimental.pallas import tpu as pltpu

_NT = (((1,), (1,)), ((), ()))


def _moe_body(x_ref, wr_ref, w1_ref, w2_ref, out_ref, scores_ref, *, tm):
    e = pl.program_id(0)
    k = pl.program_id(1)
    step = e + k

    @pl.when(step == 0)
    def _():
        logits = jax.lax.dot_general(
            x_ref[...], wr_ref[...], _NT, preferred_element_type=jnp.float32)
        m = jnp.max(logits, axis=-1, keepdims=True)
        p = jnp.exp(logits - m)
        scores_ref[...] = p / jnp.sum(p, axis=-1, keepdims=True)
        out_ref[...] = jnp.zeros_like(out_ref)

    w1c = w1_ref[0].astype(jnp.bfloat16)
    w2c = w2_ref[0].astype(jnp.bfloat16)
    t_total = x_ref.shape[0]
    for t in range(t_total // tm):
        sl = pl.ds(t * tm, tm)
        xb = x_ref[sl, :]
        h = jax.lax.dot_general(
            xb, w1c, _NT, preferred_element_type=jnp.float32)
        g = 0.5 * h
        h = g + g * jnp.tanh(g)
        scores = scores_ref[sl, :]
        lane = jax.lax.broadcasted_iota(jnp.int32, scores.shape, 1)
        s = jnp.sum(jnp.where(lane == e, scores, 0.0), axis=-1, keepdims=True)
        hb = (h * s).astype(jnp.bfloat16)
        out_ref[sl, :] += jax.lax.dot_general(
            hb, w2c, _NT, preferred_element_type=jnp.float32)


def kernel(x, Wr, W1, W2):
    B, S, H = x.shape
    E, I, _ = W1.shape
    T = B * S
    xf = x.reshape(T, H).astype(jnp.bfloat16)
    wrb = Wr.astype(jnp.bfloat16)
    K = 2 if I % 2 == 0 else 1
    C = I // K
    TM = 2048 if T % 2048 == 0 else T

    body = functools.partial(_moe_body, tm=TM)
    out = pl.pallas_call(
        body,
        grid=(E, K),
        in_specs=[
            pl.BlockSpec((T, H), lambda e, k: (0, 0)),
            pl.BlockSpec((E, H), lambda e, k: (0, 0)),
            pl.BlockSpec((1, C, H), lambda e, k: (e, k, 0)),
            pl.BlockSpec((1, H, C), lambda e, k: (e, 0, k)),
        ],
        out_specs=pl.BlockSpec((T, H), lambda e, k: (0, 0)),
        out_shape=jax.ShapeDtypeStruct((T, H), jnp.float32),
        scratch_shapes=[pltpu.VMEM((T, E), jnp.float32)],
        compiler_params=pltpu.CompilerParams(
            vmem_limit_bytes=100 * 1024 * 1024),
        interpret=False,
    )(xf, wrb, W1, W2)
    return out.reshape(B, S, H)


# speedup vs baseline: 1.0825x; 1.0825x over previous
import functools

import jax
import jax.numpy as jnp
from jax.experimental import pallas as pl
from jax.experimental.pallas import tpu as pltpu

_NT = (((1,), (1,)), ((), ()))


def _moe_body(x_ref, wr_ref, w1_ref, w2_ref, out_ref, scores_ref, *, tm):
    e = pl.program_id(0)
    k = pl.program_id(1)
    step = e + k

    @pl.when(step == 0)
    def _():
        logits = jax.lax.dot_general(
            x_ref[...], wr_ref[...], _NT, preferred_element_type=jnp.float32)
        m = jnp.max(logits, axis=-1, keepdims=True)
        p = jnp.exp(logits - m)
        scores_ref[...] = p / jnp.sum(p, axis=-1, keepdims=True)
        out_ref[...] = jnp.zeros_like(out_ref)

    w1c = w1_ref[0].astype(jnp.bfloat16)
    w2c = w2_ref[0].astype(jnp.bfloat16)
    t_total = x_ref.shape[0]
    for t in range(t_total // tm):
        sl = pl.ds(t * tm, tm)
        xb = x_ref[sl, :]
        h = jax.lax.dot_general(
            xb, w1c, _NT, preferred_element_type=jnp.float32)
        g = 0.5 * h
        h = g + g * jnp.tanh(g)
        scores = scores_ref[sl, :]
        lane = jax.lax.broadcasted_iota(jnp.int32, scores.shape, 1)
        s = jnp.sum(jnp.where(lane == e, scores, 0.0), axis=-1, keepdims=True)
        hb = h.astype(jnp.bfloat16) * s.astype(jnp.bfloat16)
        out_ref[sl, :] += jax.lax.dot_general(
            hb, w2c, _NT, preferred_element_type=jnp.float32)


def kernel(x, Wr, W1, W2):
    B, S, H = x.shape
    E, I, _ = W1.shape
    T = B * S
    xf = x.reshape(T, H).astype(jnp.bfloat16)
    wrb = Wr.astype(jnp.bfloat16)
    K = 2 if I % 2 == 0 else 1
    C = I // K
    TM = 1024 if T % 1024 == 0 else T

    body = functools.partial(_moe_body, tm=TM)
    out = pl.pallas_call(
        body,
        grid=(E, K),
        in_specs=[
            pl.BlockSpec((T, H), lambda e, k: (0, 0)),
            pl.BlockSpec((E, H), lambda e, k: (0, 0)),
            pl.BlockSpec((1, C, H), lambda e, k: (e, k, 0)),
            pl.BlockSpec((1, H, C), lambda e, k: (e, 0, k)),
        ],
        out_specs=pl.BlockSpec((T, H), lambda e, k: (0, 0)),
        out_shape=jax.ShapeDtypeStruct((T, H), jnp.float32),
        scratch_shapes=[pltpu.VMEM((T, E), jnp.float32)],
        compiler_params=pltpu.CompilerParams(
            vmem_limit_bytes=100 * 1024 * 1024),
        interpret=False,
    )(xf, wrb, W1, W2)
    return out.reshape(B, S, H)


# speedup vs baseline: 1.0850x; 1.0022x over previous
import functools

import jax
import jax.numpy as jnp
from jax.experimental import pallas as pl
from jax.experimental.pallas import tpu as pltpu

_NT = (((1,), (1,)), ((), ()))


def _moe_body(x_ref, wr_ref, w1_ref, w2_ref, out_ref, scores_ref, *, tm):
    e = pl.program_id(0)
    k = pl.program_id(1)
    step = e + k

    @pl.when(step == 0)
    def _():
        logits = jax.lax.dot_general(
            x_ref[...], wr_ref[...], _NT, preferred_element_type=jnp.float32)
        m = jnp.max(logits, axis=-1, keepdims=True)
        p = jnp.exp(logits - m)
        scores_ref[...] = p / jnp.sum(p, axis=-1, keepdims=True)
        out_ref[...] = jnp.zeros_like(out_ref)

    w1c = w1_ref[0].astype(jnp.bfloat16)
    w2c = w2_ref[0].astype(jnp.bfloat16)
    t_total = x_ref.shape[0]
    hbs = []
    for t in range(t_total // tm):
        sl = pl.ds(t * tm, tm)
        xb = x_ref[sl, :]
        h = jax.lax.dot_general(
            xb, w1c, _NT, preferred_element_type=jnp.float32)
        g = 0.5 * h
        h = g + g * jnp.tanh(g)
        scores = scores_ref[sl, :]
        lane = jax.lax.broadcasted_iota(jnp.int32, scores.shape, 1)
        s = jnp.sum(jnp.where(lane == e, scores, 0.0), axis=-1, keepdims=True)
        hbs.append(h.astype(jnp.bfloat16) * s.astype(jnp.bfloat16))
    for t in range(t_total // tm):
        sl = pl.ds(t * tm, tm)
        out_ref[sl, :] += jax.lax.dot_general(
            hbs[t], w2c, _NT, preferred_element_type=jnp.float32)


def kernel(x, Wr, W1, W2):
    B, S, H = x.shape
    E, I, _ = W1.shape
    T = B * S
    xf = x.reshape(T, H).astype(jnp.bfloat16)
    wrb = Wr.astype(jnp.bfloat16)
    K = 2 if I % 2 == 0 else 1
    C = I // K
    TM = 1024 if T % 1024 == 0 else T

    body = functools.partial(_moe_body, tm=TM)
    out = pl.pallas_call(
        body,
        grid=(E, K),
        in_specs=[
            pl.BlockSpec((T, H), lambda e, k: (0, 0)),
            pl.BlockSpec((E, H), lambda e, k: (0, 0)),
            pl.BlockSpec((1, C, H), lambda e, k: (e, k, 0)),
            pl.BlockSpec((1, H, C), lambda e, k: (e, 0, k)),
        ],
        out_specs=pl.BlockSpec((T, H), lambda e, k: (0, 0)),
        out_shape=jax.ShapeDtypeStruct((T, H), jnp.float32),
        scratch_shapes=[pltpu.VMEM((T, E), jnp.float32)],
        compiler_params=pltpu.CompilerParams(
            vmem_limit_bytes=100 * 1024 * 1024),
        interpret=False,
    )(xf, wrb, W1, W2)
    return out.reshape(B, S, H)
